# baseline (speedup 1.0000x reference)
import functools

import jax
import jax.numpy as jnp
from jax import lax
from jax.experimental import pallas as pl
from jax.experimental.pallas import tpu as pltpu
from jax.experimental.pallas import tpu_sc as plsc

BATCH = 4096
SEQ = 200
EMB = 64
LANES = 16
NCOL = EMB // LANES

NC = 2
NS = 16
NW = NC * NS
B_PER_W = BATCH // NW
CHUNKS_PER_B = 2
CHUNK = SEQ // CHUNKS_PER_B
ROWS_PER_W = B_PER_W * CHUNKS_PER_B

_NEG = float(jnp.finfo(jnp.float32).min)
_UNROLL = 20


def _reduce_chunk(buf):
    def body(it, accs):
        s0 = it * _UNROLL
        for u in range(_UNROLL):
            accs = tuple(
                jnp.maximum(a, buf[s0 + u, pl.ds(LANES * j, LANES)])
                for j, a in enumerate(accs)
            )
        return accs

    init = tuple(jnp.full((LANES,), _NEG, jnp.float32) for _ in range(NCOL))
    return lax.fori_loop(0, CHUNK // _UNROLL, body, init)


@functools.partial(
    pl.kernel,
    out_type=jax.ShapeDtypeStruct((BATCH, EMB), jnp.float32),
    mesh=plsc.VectorSubcoreMesh(core_axis_name="c", subcore_axis_name="s"),
    compiler_params=pltpu.CompilerParams(use_tc_tiling_on_sc=False),
    scratch_types=[
        pltpu.VMEM((ROWS_PER_W, CHUNK), jnp.int32),
        pltpu.VMEM((CHUNK, EMB), jnp.float32),
        pltpu.VMEM((CHUNK, EMB), jnp.float32),
        pltpu.VMEM((CHUNK, EMB), jnp.float32),
        pltpu.VMEM((CHUNK, EMB), jnp.float32),
        pltpu.VMEM((CHUNK, EMB), jnp.float32),
        pltpu.VMEM((CHUNK, EMB), jnp.float32),
        pltpu.VMEM((CHUNK, EMB), jnp.float32),
        pltpu.VMEM((CHUNK, EMB), jnp.float32),
        pltpu.VMEM((B_PER_W, EMB), jnp.float32),
        pltpu.SemaphoreType.DMA,
        pltpu.SemaphoreType.DMA,
        pltpu.SemaphoreType.DMA,
        pltpu.SemaphoreType.DMA,
        pltpu.SemaphoreType.DMA,
        pltpu.SemaphoreType.DMA,
        pltpu.SemaphoreType.DMA,
        pltpu.SemaphoreType.DMA,
    ],
)
def _bow_encode(idx_hbm, table_hbm, out_hbm, idx_v,
                buf0, buf1, buf2, buf3, buf4, buf5, buf6, buf7, out_v,
                sem0, sem1, sem2, sem3, sem4, sem5, sem6, sem7):
    wid = lax.axis_index("s") * NC + lax.axis_index("c")
    base = wid * ROWS_PER_W
    bufs = (buf0, buf1, buf2, buf3, buf4, buf5, buf6, buf7)
    sems = (sem0, sem1, sem2, sem3, sem4, sem5, sem6, sem7)
    nbuf = len(bufs)

    pltpu.sync_copy(idx_hbm.at[pl.ds(base, ROWS_PER_W), :], idx_v)

    for q in range(nbuf):
        pltpu.async_copy(table_hbm.at[idx_v.at[q]], bufs[q], sems[q])

    def gbody(h, carry):
        accs = []
        for q in range(nbuf):
            r = nbuf * h + q
            pltpu.make_async_copy(table_hbm.at[idx_v.at[r]], bufs[q], sems[q]).wait()
            accs.append(_reduce_chunk(bufs[q]))

            @pl.when(h < ROWS_PER_W // nbuf - 1)
            def _():
                pltpu.async_copy(table_hbm.at[idx_v.at[r + nbuf]], bufs[q], sems[q])

        for j in range(NCOL):
            for k in range(nbuf // 2):
                out_v[(nbuf // 2) * h + k, pl.ds(LANES * j, LANES)] = jnp.maximum(
                    accs[2 * k][j], accs[2 * k + 1][j]
                )
        return carry

    lax.fori_loop(0, ROWS_PER_W // nbuf, gbody, 0)

    pltpu.sync_copy(out_v, out_hbm.at[pl.ds(wid * B_PER_W, B_PER_W), :])


@jax.jit
def kernel(input, emb_weight):
    idx = input.astype(jnp.int32).reshape(BATCH * CHUNKS_PER_B, CHUNK)
    return _bow_encode(idx, emb_weight)


# speedup vs baseline: 1.0230x; 1.0230x over previous
import functools

import jax
import jax.numpy as jnp
from jax import lax
from jax.experimental import pallas as pl
from jax.experimental.pallas import tpu as pltpu
from jax.experimental.pallas import tpu_sc as plsc

BATCH = 4096
SEQ = 200
EMB = 64
LANES = 16
NCOL = EMB // LANES

NC = 2
NS = 16
NW = NC * NS
B_PER_W = BATCH // NW
CHUNKS_PER_B = 2
CHUNK = SEQ // CHUNKS_PER_B
ROWS_PER_W = B_PER_W * CHUNKS_PER_B

_NEG = float(jnp.finfo(jnp.float32).min)
_UNROLL = 20


def _reduce_chunk(buf):
    def body(it, accs):
        s0 = it * _UNROLL
        for u in range(_UNROLL):
            accs = tuple(
                jnp.maximum(a, buf[s0 + u, pl.ds(LANES * j, LANES)])
                for j, a in enumerate(accs)
            )
        return accs

    init = tuple(jnp.full((LANES,), _NEG, jnp.float32) for _ in range(NCOL))
    return lax.fori_loop(0, CHUNK // _UNROLL, body, init)


@functools.partial(
    pl.kernel,
    out_type=jax.ShapeDtypeStruct((BATCH, EMB), jnp.float32),
    mesh=plsc.VectorSubcoreMesh(core_axis_name="c", subcore_axis_name="s"),
    compiler_params=pltpu.CompilerParams(use_tc_tiling_on_sc=False),
    scratch_types=[
        pltpu.VMEM((ROWS_PER_W, CHUNK), jnp.int32),
        pltpu.VMEM((CHUNK, EMB), jnp.float32),
        pltpu.VMEM((CHUNK, EMB), jnp.float32),
        pltpu.VMEM((CHUNK, EMB), jnp.float32),
        pltpu.VMEM((CHUNK, EMB), jnp.float32),
        pltpu.VMEM((B_PER_W, EMB), jnp.float32),
        pltpu.SemaphoreType.DMA,
        pltpu.SemaphoreType.DMA,
        pltpu.SemaphoreType.DMA,
        pltpu.SemaphoreType.DMA,
    ],
)
def _bow_encode(idx_hbm, table_hbm, out_hbm, idx_v,
                buf0, buf1, buf2, buf3, out_v, sem0, sem1, sem2, sem3):
    wid = lax.axis_index("s") * NC + lax.axis_index("c")
    base = wid * ROWS_PER_W
    bufs = (buf0, buf1, buf2, buf3)
    sems = (sem0, sem1, sem2, sem3)
    nbuf = len(bufs)

    pltpu.sync_copy(idx_hbm.at[pl.ds(base, ROWS_PER_W), :], idx_v)

    for q in range(nbuf):
        pltpu.async_copy(table_hbm.at[idx_v.at[q]], bufs[q], sems[q])

    def gbody(h, carry):
        accs = []
        for q in range(nbuf):
            r = nbuf * h + q
            pltpu.make_async_copy(table_hbm.at[idx_v.at[r]], bufs[q], sems[q]).wait()
            accs.append(_reduce_chunk(bufs[q]))

            @pl.when(h < ROWS_PER_W // nbuf - 1)
            def _():
                pltpu.async_copy(table_hbm.at[idx_v.at[r + nbuf]], bufs[q], sems[q])

        for j in range(NCOL):
            for k in range(nbuf // 2):
                out_v[(nbuf // 2) * h + k, pl.ds(LANES * j, LANES)] = jnp.maximum(
                    accs[2 * k][j], accs[2 * k + 1][j]
                )
        return carry

    lax.fori_loop(0, ROWS_PER_W // nbuf, gbody, 0)

    pltpu.sync_copy(out_v, out_hbm.at[pl.ds(wid * B_PER_W, B_PER_W), :])


@jax.jit
def kernel(input, emb_weight):
    idx = input.astype(jnp.int32).reshape(BATCH * CHUNKS_PER_B, CHUNK)
    return _bow_encode(idx, emb_weight)
